# Initial kernel scaffold; baseline (speedup 1.0000x reference)
import functools

import jax
import jax.numpy as jnp
from jax import lax
from jax.experimental import pallas as pl
from jax.experimental.pallas import tpu as pltpu
from jax.experimental.pallas import tpu_sc as plsc

_D = 32
_B = 4096 * 200

_info = plsc.get_sparse_core_info()
_NC, _NS = _info.num_cores, _info.num_subcores
_NW = _NC * _NS
_B_PER_W = _B // _NW
_CHUNK = 1024
_N_CHUNKS = _B_PER_W // _CHUNK

_mesh = plsc.VectorSubcoreMesh(core_axis_name="c", subcore_axis_name="s")


@functools.partial(
    pl.kernel,
    mesh=_mesh,
    out_type=jax.ShapeDtypeStruct((_B, _D), jnp.float32),
    scratch_types=[
        pltpu.VMEM((_CHUNK,), jnp.int32),
        pltpu.VMEM((_CHUNK, _D), jnp.float32),
        pltpu.SemaphoreType.DMA,
    ],
)
def _embed(idx_hbm, table_hbm, out_hbm, idx_v, rows_v, sem):
    wid = lax.axis_index("s") * _NC + lax.axis_index("c")
    base = wid * _B_PER_W

    def body(j, carry):
        off = base + j * _CHUNK
        pltpu.sync_copy(idx_hbm.at[pl.ds(off, _CHUNK)], idx_v)
        pltpu.async_copy(table_hbm.at[idx_v], rows_v, sem).wait()
        pltpu.sync_copy(rows_v, out_hbm.at[pl.ds(off, _CHUNK)])
        return carry

    lax.fori_loop(0, _N_CHUNKS, body, 0)


def kernel(seq, table):
    flat = seq.reshape(-1)
    out = _embed(flat, table)
    return out.reshape(seq.shape[0], seq.shape[1], _D)


# speedup vs baseline: 1.5013x; 1.5013x over previous
import functools

import jax
import jax.numpy as jnp
from jax import lax
from jax.experimental import pallas as pl
from jax.experimental.pallas import tpu as pltpu
from jax.experimental.pallas import tpu_sc as plsc

_D = 32
_B = 4096 * 200

_info = plsc.get_sparse_core_info()
_NC, _NS = _info.num_cores, _info.num_subcores
_NW = _NC * _NS
_B_PER_W = _B // _NW
_CHUNK = 1280
_N_CHUNKS = _B_PER_W // _CHUNK

_mesh = plsc.VectorSubcoreMesh(core_axis_name="c", subcore_axis_name="s")


@functools.partial(
    pl.kernel,
    mesh=_mesh,
    out_type=jax.ShapeDtypeStruct((_B, _D), jnp.float32),
    scratch_types=[
        pltpu.VMEM((_B_PER_W,), jnp.int32),
        pltpu.VMEM((_CHUNK, _D), jnp.float32),
        pltpu.VMEM((_CHUNK, _D), jnp.float32),
        pltpu.SemaphoreType.DMA,
        pltpu.SemaphoreType.DMA,
        pltpu.SemaphoreType.DMA,
        pltpu.SemaphoreType.DMA,
    ],
    compiler_params=pltpu.CompilerParams(use_tc_tiling_on_sc=False),
)
def _embed(idx_hbm, table_hbm, out_hbm, idx_v, buf0, buf1, gs0, gs1, ss0, ss1):
    wid = lax.axis_index("s") * _NC + lax.axis_index("c")
    base = wid * _B_PER_W
    bufs = (buf0, buf1)
    gsems = (gs0, gs1)
    ssems = (ss0, ss1)

    pltpu.sync_copy(idx_hbm.at[pl.ds(base, _B_PER_W)], idx_v)

    def idx_slice(j):
        return idx_v.at[pl.ds(j * _CHUNK, _CHUNK)]

    def out_slice(j):
        return out_hbm.at[pl.ds(base + j * _CHUNK, _CHUNK)]

    def start_gather(j, b):
        pltpu.async_copy(table_hbm.at[idx_slice(j)], bufs[b], gsems[b])

    def wait_gather(j, b):
        pltpu.make_async_copy(table_hbm.at[idx_slice(j)], bufs[b], gsems[b]).wait()

    def start_store(j, b):
        pltpu.async_copy(bufs[b], out_slice(j), ssems[b])

    def wait_store(j, b):
        pltpu.make_async_copy(bufs[b], out_slice(j), ssems[b]).wait()

    start_gather(0, 0)

    def outer(i, carry):
        j0 = i * 2

        @pl.when(j0 >= 1)
        def _():
            wait_store(j0 - 1, 1)

        start_gather(j0 + 1, 1)
        wait_gather(j0, 0)
        start_store(j0, 0)

        @pl.when(j0 + 2 < _N_CHUNKS)
        def _():
            wait_store(j0, 0)
            start_gather(j0 + 2, 0)

        wait_gather(j0 + 1, 1)
        start_store(j0 + 1, 1)
        return carry

    lax.fori_loop(0, _N_CHUNKS // 2, outer, 0)

    wait_store(_N_CHUNKS - 2, 0)
    wait_store(_N_CHUNKS - 1, 1)


def kernel(seq, table):
    flat = seq.reshape(-1)
    out = _embed(flat, table)
    return out.reshape(seq.shape[0], seq.shape[1], _D)


# speedup vs baseline: 1.5014x; 1.0001x over previous
import functools

import jax
import jax.numpy as jnp
from jax import lax
from jax.experimental import pallas as pl
from jax.experimental.pallas import tpu as pltpu
from jax.experimental.pallas import tpu_sc as plsc

_D = 32
_B = 4096 * 200

_info = plsc.get_sparse_core_info()
_NC, _NS = _info.num_cores, _info.num_subcores
_NW = _NC * _NS
_B_PER_W = _B // _NW
_CHUNK = 640
_N_CHUNKS = _B_PER_W // _CHUNK
_NBUF = 4

_mesh = plsc.VectorSubcoreMesh(core_axis_name="c", subcore_axis_name="s")


@functools.partial(
    pl.kernel,
    mesh=_mesh,
    out_type=jax.ShapeDtypeStruct((_B, _D), jnp.float32),
    scratch_types=[
        pltpu.VMEM((_B_PER_W,), jnp.int32),
        [pltpu.VMEM((_CHUNK, _D), jnp.float32) for _ in range(_NBUF)],
        [pltpu.SemaphoreType.DMA for _ in range(_NBUF)],
        [pltpu.SemaphoreType.DMA for _ in range(_NBUF)],
    ],
    compiler_params=pltpu.CompilerParams(use_tc_tiling_on_sc=False),
)
def _embed(idx_hbm, table_hbm, out_hbm, idx_v, bufs, gsems, ssems):
    wid = lax.axis_index("s") * _NC + lax.axis_index("c")
    base = wid * _B_PER_W

    pltpu.sync_copy(idx_hbm.at[pl.ds(base, _B_PER_W)], idx_v)

    def idx_slice(j):
        return idx_v.at[pl.ds(j * _CHUNK, _CHUNK)]

    def out_slice(j):
        return out_hbm.at[pl.ds(base + j * _CHUNK, _CHUNK)]

    def start_gather(j, b):
        pltpu.async_copy(table_hbm.at[idx_slice(j)], bufs[b], gsems[b])

    def wait_gather(j, b):
        pltpu.make_async_copy(table_hbm.at[idx_slice(j)], bufs[b], gsems[b]).wait()

    def start_store(j, b):
        pltpu.async_copy(bufs[b], out_slice(j), ssems[b])

    def wait_store(j, b):
        pltpu.make_async_copy(bufs[b], out_slice(j), ssems[b]).wait()

    for j in range(_NBUF - 1):
        start_gather(j, j)

    def outer(i, carry):
        j0 = i * _NBUF
        for b in range(_NBUF):
            j = j0 + b
            jn = j + _NBUF - 1
            bn = (b - 1) % _NBUF

            @pl.when(jn < _N_CHUNKS)
            def _(j=j, jn=jn, bn=bn):
                @pl.when(j >= 1)
                def _():
                    wait_store(j - 1, bn)

                start_gather(jn, bn)

            wait_gather(j, b)
            start_store(j, b)
        return carry

    lax.fori_loop(0, _N_CHUNKS // _NBUF, outer, 0)

    for k in range(_NBUF):
        j = _N_CHUNKS - _NBUF + k
        wait_store(j, j % _NBUF)


def kernel(seq, table):
    flat = seq.reshape(-1)
    out = _embed(flat, table)
    return out.reshape(seq.shape[0], seq.shape[1], _D)
